# Initial kernel scaffold; baseline (speedup 1.0000x reference)
import jax
import jax.numpy as jnp
from jax.experimental import pallas as pl


def kernel(x, edge_index, graph_ids, eps0, W0a, b0a, W0b, b0b, gmlp0, bmlp0, gbn0, bbn0, eps1, W1a, b1a, gmlp1, bmlp1, gbn1, bbn1, Wr1, br1, Wr2, br2):
    raise NotImplementedError("write your pallas kernel here")


# speedup vs baseline: 5.9697x; 5.9697x over previous
import functools

import jax
import jax.numpy as jnp
from jax import lax
from jax.experimental import pallas as pl
from jax.experimental.pallas import tpu as pltpu
from jax.experimental.pallas import tpu_sc as plsc

N = 10000
E = 320000
D = 128
G = 64

NC = 2
NS = 16
NW = NC * NS

B = 128
NCH = E // B
KMAX = (NCH + NW - 1) // NW

ROWS_PER_TILE = 640
NPAD = ROWS_PER_TILE * NS


def _seg_body(src_hbm, dst_hbm, x_hbm, out_hbm, idx_s, idx_d, rows, acc, sem):
    cid = lax.axis_index("c")
    sid = lax.axis_index("s")
    wid = sid * NC + cid

    zeros16 = jnp.zeros((16,), jnp.float32)

    def zbody(i, carry):
        for j in range(D // 16):
            rows[i, pl.ds(j * 16, 16)] = zeros16
        return carry

    lax.fori_loop(0, B, zbody, 0)
    rbase = sid * ROWS_PER_TILE
    for z in range(ROWS_PER_TILE // B):
        pltpu.sync_copy(rows, acc.at[pl.ds(rbase + z * B, B)])
    plsc.subcore_barrier()

    def chunk(k, carry):
        c = wid + k * NW

        @pl.when(c < NCH)
        def _():
            off = c * B
            pltpu.sync_copy(src_hbm.at[pl.ds(off, B)], idx_s)
            pltpu.sync_copy(dst_hbm.at[pl.ds(off, B)], idx_d)
            pltpu.async_copy(x_hbm.at[idx_s], rows, sem).wait()
            pltpu.sync_copy(rows, acc.at[idx_d], add=True)

        return carry

    lax.fori_loop(0, KMAX, chunk, 0)
    plsc.subcore_barrier()

    pltpu.sync_copy(
        acc.at[pl.ds(rbase, ROWS_PER_TILE)],
        out_hbm.at[cid, pl.ds(rbase, ROWS_PER_TILE)],
    )


@functools.cache
def _make_segsum():
    return pl.kernel(
        _seg_body,
        out_type=jax.ShapeDtypeStruct((NC, NPAD, D), jnp.float32),
        mesh=plsc.VectorSubcoreMesh(
            core_axis_name="c", subcore_axis_name="s", num_cores=NC, num_subcores=NS
        ),
        scratch_types=[
            pltpu.VMEM((B,), jnp.int32),
            pltpu.VMEM((B,), jnp.int32),
            pltpu.VMEM((B, D), jnp.float32),
            pltpu.VMEM_SHARED((NPAD, D), jnp.float32),
            pltpu.SemaphoreType.DMA,
        ],
    )


def _segsum(src, dst, x):
    return _make_segsum()(src, dst, x)


def _dotT(h, w):
    return lax.dot_general(
        h, w, (((1,), (1,)), ((), ())),
        preferred_element_type=jnp.float32,
    )


def _bn(h, g, b):
    mu = jnp.mean(h, axis=0, keepdims=True)
    var = jnp.mean((h - mu) ** 2, axis=0, keepdims=True)
    return (h - mu) * lax.rsqrt(var + 1e-5) * g + b


def _mlp0_body(eps_ref, x_ref, a_ref, wa_ref, ba_ref, wb_ref, bb_ref,
               gm_ref, bm_ref, gb_ref, bb2_ref, h_ref):
    s = 1.0 + eps_ref[0]
    h = s * x_ref[...] + a_ref[0, :N, :] + a_ref[1, :N, :]
    h = jnp.maximum(_dotT(h, wa_ref[...]) + ba_ref[...], 0.0)
    h = jnp.maximum(_dotT(h, wb_ref[...]) + bb_ref[...], 0.0)
    h = _bn(h, gm_ref[...], bm_ref[...])
    h = jnp.maximum(_bn(h, gb_ref[...], bb2_ref[...]), 0.0)
    h_ref[...] = h


def _tail_body(eps_ref, h_ref, a_ref, gid_ref, w1_ref, b1_ref,
               gm_ref, bm_ref, gb_ref, bb_ref,
               wr1_ref, br1_ref, wr2_ref, br2_ref, o_ref, h2_ref):
    s = 1.0 + eps_ref[0]
    h2 = s * h_ref[...] + a_ref[0, :N, :] + a_ref[1, :N, :]
    h2 = jnp.maximum(_dotT(h2, w1_ref[...]) + b1_ref[...], 0.0)
    h2 = _bn(h2, gm_ref[...], bm_ref[...])
    h2 = jnp.maximum(_bn(h2, gb_ref[...], bb_ref[...]), 0.0)
    h2_ref[...] = h2
    gids = gid_ref[...]
    P = (lax.broadcasted_iota(jnp.int32, (G, N), 0) == gids).astype(jnp.float32)
    sums = lax.dot_general(
        P, h2, (((1,), (0,)), ((), ())),
        preferred_element_type=jnp.float32,
    )
    cnt = jnp.sum(P, axis=1, keepdims=True)
    hg = sums / jnp.maximum(cnt, 1.0)
    r = jnp.maximum(_dotT(hg, wr1_ref[...]) + br1_ref[...], 0.0)
    o_ref[...] = _dotT(r, wr2_ref[...]) + br2_ref[...]


_smem = pl.BlockSpec(memory_space=pltpu.SMEM)
_vmem = pl.BlockSpec(memory_space=pltpu.VMEM)

_mlp0 = pl.pallas_call(
    _mlp0_body,
    out_shape=jax.ShapeDtypeStruct((N, D), jnp.float32),
    in_specs=[_smem] + [_vmem] * 10,
    out_specs=_vmem,
)

_tail = pl.pallas_call(
    _tail_body,
    out_shape=[
        jax.ShapeDtypeStruct((G, D), jnp.float32),
        jax.ShapeDtypeStruct((N, D), jnp.float32),
    ],
    in_specs=[_smem] + [_vmem] * 13,
    out_specs=[_vmem, _vmem],
)


def kernel(x, edge_index, graph_ids, eps0, W0a, b0a, W0b, b0b, gmlp0, bmlp0,
           gbn0, bbn0, eps1, W1a, b1a, gmlp1, bmlp1, gbn1, bbn1, Wr1, br1,
           Wr2, br2):
    src = edge_index[0]
    dst = edge_index[1]
    r = lambda v: v.reshape(1, D)

    a0 = _segsum(src, dst, x)
    h = _mlp0(eps0.reshape(1), x, a0, W0a, r(b0a), W0b, r(b0b),
              r(gmlp0), r(bmlp0), r(gbn0), r(bbn0))
    a1 = _segsum(src, dst, h)
    out, h2 = _tail(eps1.reshape(1), h, a1, graph_ids.reshape(1, N), W1a,
                    r(b1a), r(gmlp1), r(bmlp1), r(gbn1), r(bbn1),
                    Wr1, r(br1), Wr2, r(br2))
    return (out, h2)
